# baseline (speedup 1.0000x reference)
import functools

import jax
import jax.numpy as jnp
from jax import lax
from jax.experimental import pallas as pl
from jax.experimental.pallas import tpu as pltpu
from jax.experimental.pallas import tpu_sc as plsc

_C = 64
_M = 256
_N = 256
_T = 2048
_NC = 2
_NS = 16
_NW = _NC * _NS
_CHUNK = _T // _NW
_TT = 64
_NT = _T // _TT + _C // 2
_PAD = _NT * _TT

_LANES = 16
_IOTA = None


def _vgather(v, idx):
    return lax.gather(
        v,
        idx[:, None],
        lax.GatherDimensionNumbers(
            offset_dims=(), collapsed_slice_dims=(0,), start_index_map=(0,)),
        slice_sizes=(1,),
        mode=lax.GatherScatterMode.PROMISE_IN_BOUNDS,
    )


def _lane_splat(v, l):
    return _vgather(v, jnp.full((_LANES,), l, jnp.int32))


def _worker_id():
    return lax.axis_index("s") * _NC + lax.axis_index("c")


def _hist_body(inds_hbm, r_hbm, ranks_hbm, k_v, cnt_v, ranks_v):
    wid = _worker_id()
    pltpu.sync_copy(inds_hbm.at[pl.ds(wid * _CHUNK, _CHUNK)], k_v)
    iota = lax.iota(jnp.int32, _LANES)
    for q in range(_C // _LANES):
        cnt_v[pl.ds(q * _LANES, _LANES)] = jnp.zeros((_LANES,), jnp.int32)
    for q in range(_CHUNK // _LANES):
        k = k_v[pl.ds(q * _LANES, _LANES)]
        rk = jnp.zeros((_LANES,), jnp.int32)
        cnt = jnp.zeros((_LANES,), jnp.int32)
        for l in range(_LANES):
            eq = (k == _lane_splat(k, l)).astype(jnp.int32)
            rk = rk + jnp.where(iota > l, eq, 0)
            cnt = cnt + eq
        prev = plsc.load_gather(cnt_v, [k])
        ranks_v[pl.ds(q * _LANES, _LANES)] = prev + rk
        plsc.addupdate_scatter(cnt_v, [k], cnt, mask=(rk == cnt - 1))
    pltpu.sync_copy(cnt_v, r_hbm.at[wid])
    pltpu.sync_copy(ranks_v, ranks_hbm.at[pl.ds(wid * _CHUNK, _CHUNK)])


def _route_body(inds_hbm, x_hbm, r_hbm, ranks_hbm,
                slot_hbm, texp_hbm, xs_hbm,
                rall_v, btab_v, ttab_v, k_v, rk_v, slot_v, texp_v, xr_v, sem):
    wid = _worker_id()
    iota = lax.iota(jnp.int32, _LANES)
    nq = _C // _LANES
    pltpu.sync_copy(r_hbm, rall_v)
    start = [jnp.zeros((_LANES,), jnp.int32) for _ in range(nq)]
    total = [jnp.zeros((_LANES,), jnp.int32) for _ in range(nq)]
    for w in range(_NW):
        mw = jnp.where(w < wid, jnp.int32(1), jnp.int32(0))
        for q in range(nq):
            v = rall_v[w, pl.ds(q * _LANES, _LANES)]
            total[q] = total[q] + v
            start[q] = start[q] + v * mw
    carry = jnp.zeros((_LANES,), jnp.int32)
    for q in range(nq):
        tiles = (total[q] + (_TT - 1)) >> 6
        incl = jnp.cumsum(tiles)
        excl = incl - tiles + carry
        carry = carry + _lane_splat(incl, _LANES - 1)
        ttab_v[pl.ds(q * _LANES, _LANES)] = excl
        btab_v[pl.ds(q * _LANES, _LANES)] = excl * _TT + start[q]
    pltpu.sync_copy(inds_hbm.at[pl.ds(wid * _CHUNK, _CHUNK)], k_v)
    pltpu.sync_copy(ranks_hbm.at[pl.ds(wid * _CHUNK, _CHUNK)], rk_v)
    for q in range(_CHUNK // _LANES):
        k = k_v[pl.ds(q * _LANES, _LANES)]
        off = plsc.load_gather(btab_v, [k])
        slot_v[pl.ds(q * _LANES, _LANES)] = off + rk_v[pl.ds(q * _LANES, _LANES)]
    pltpu.sync_copy(slot_v, slot_hbm.at[pl.ds(wid * _CHUNK, _CHUNK)])
    pltpu.sync_copy(x_hbm.at[pl.ds(wid * _CHUNK, _CHUNK)], xr_v)
    pltpu.async_copy(xr_v, xs_hbm.at[slot_v], sem).wait()

    @pl.when(wid == 0)
    def _():
        for j in range(_NT // _LANES):
            jv = iota + j * _LANES
            e = jnp.full((_LANES,), -1, jnp.int32)
            for c in range(_C):
                spl = plsc.load_gather(
                    ttab_v, [jnp.full((_LANES,), c, jnp.int32)])
                e = e + (spl <= jv).astype(jnp.int32)
            texp_v[pl.ds(j * _LANES, _LANES)] = e
        pltpu.sync_copy(texp_v, texp_hbm)


def _mm_body(texp_ref, xs_ref, w_ref, b_ref, y_ref):
    del texp_ref
    y_ref[...] = jnp.dot(
        xs_ref[...].astype(jnp.bfloat16), w_ref[0],
        preferred_element_type=jnp.float32) + b_ref[0]


def _unsort_body(slot_hbm, ys_hbm, out_hbm, slot_v, rows_v, sem):
    wid = _worker_id()
    pltpu.sync_copy(slot_hbm.at[pl.ds(wid * _CHUNK, _CHUNK)], slot_v)
    pltpu.async_copy(ys_hbm.at[slot_v], rows_v, sem).wait()
    pltpu.sync_copy(rows_v, out_hbm.at[pl.ds(wid * _CHUNK, _CHUNK)])


def kernel(x, inds, w, b):
    inds32 = inds.astype(jnp.int32)
    wb = w.astype(jnp.bfloat16)
    mesh = plsc.VectorSubcoreMesh(
        core_axis_name="c", subcore_axis_name="s",
        num_cores=_NC, num_subcores=_NS)

    hist = pl.kernel(
        _hist_body,
        out_type=(
            jax.ShapeDtypeStruct((_NW, _C), jnp.int32),
            jax.ShapeDtypeStruct((_T,), jnp.int32),
        ),
        mesh=mesh,
        compiler_params=pltpu.CompilerParams(needs_layout_passes=False),
        scratch_types=[
            pltpu.VMEM((_CHUNK,), jnp.int32),
            pltpu.VMEM((_C,), jnp.int32),
            pltpu.VMEM((_CHUNK,), jnp.int32),
        ],
    )
    r_tab, ranks = hist(inds32)

    route = pl.kernel(
        _route_body,
        out_type=(
            jax.ShapeDtypeStruct((_T,), jnp.int32),
            jax.ShapeDtypeStruct((_NT,), jnp.int32),
            jax.ShapeDtypeStruct((_PAD, _M), jnp.float32),
        ),
        mesh=mesh,
        compiler_params=pltpu.CompilerParams(needs_layout_passes=False),
        scratch_types=[
            pltpu.VMEM((_NW, _C), jnp.int32),
            pltpu.VMEM((_C,), jnp.int32),
            pltpu.VMEM((_C,), jnp.int32),
            pltpu.VMEM((_CHUNK,), jnp.int32),
            pltpu.VMEM((_CHUNK,), jnp.int32),
            pltpu.VMEM((_CHUNK,), jnp.int32),
            pltpu.VMEM((_NT,), jnp.int32),
            pltpu.VMEM((_CHUNK, _M), jnp.float32),
            pltpu.SemaphoreType.DMA,
        ],
    )
    slot, texp, xs = route(inds32, x, r_tab, ranks)

    ys = pl.pallas_call(
        _mm_body,
        grid_spec=pltpu.PrefetchScalarGridSpec(
            num_scalar_prefetch=1,
            grid=(_NT,),
            in_specs=[
                pl.BlockSpec((_TT, _M), lambda i, te: (i, 0)),
                pl.BlockSpec((1, _M, _N), lambda i, te: (te[i], 0, 0)),
                pl.BlockSpec((1, 1, _N), lambda i, te: (te[i], 0, 0)),
            ],
            out_specs=pl.BlockSpec((_TT, _N), lambda i, te: (i, 0)),
        ),
        out_shape=jax.ShapeDtypeStruct((_PAD, _N), jnp.float32),
    )(texp, xs, wb, b)

    unsort = pl.kernel(
        _unsort_body,
        out_type=jax.ShapeDtypeStruct((_T, _N), jnp.float32),
        mesh=mesh,
        compiler_params=pltpu.CompilerParams(needs_layout_passes=False),
        scratch_types=[
            pltpu.VMEM((_CHUNK,), jnp.int32),
            pltpu.VMEM((_CHUNK, _N), jnp.float32),
            pltpu.SemaphoreType.DMA,
        ],
    )
    return unsort(slot, ys)


# speedup vs baseline: 1.4062x; 1.4062x over previous
import functools

import jax
import jax.numpy as jnp
from jax import lax
from jax.experimental import pallas as pl
from jax.experimental.pallas import tpu as pltpu
from jax.experimental.pallas import tpu_sc as plsc

_C = 64
_M = 256
_N = 256
_T = 2048
_NC = 2
_NS = 16
_NW = _NC * _NS
_CHUNK = _T // _NW
_TT = 64
_NT = _T // _TT + _C // 2
_PAD = _NT * _TT

_LANES = 16
_IOTA = None


def _vgather(v, idx):
    return lax.gather(
        v,
        idx[:, None],
        lax.GatherDimensionNumbers(
            offset_dims=(), collapsed_slice_dims=(0,), start_index_map=(0,)),
        slice_sizes=(1,),
        mode=lax.GatherScatterMode.PROMISE_IN_BOUNDS,
    )


def _lane_splat(v, l):
    return _vgather(v, jnp.full((_LANES,), l, jnp.int32))


def _worker_id():
    return lax.axis_index("s") * _NC + lax.axis_index("c")


def _hist_body(inds_hbm, r_hbm, ranks_hbm, k_v, cnt_v, ranks_v):
    wid = _worker_id()
    pltpu.sync_copy(inds_hbm.at[pl.ds(wid * _CHUNK, _CHUNK)], k_v)
    iota = lax.iota(jnp.int32, _LANES)
    for q in range(_C // _LANES):
        cnt_v[pl.ds(q * _LANES, _LANES)] = jnp.zeros((_LANES,), jnp.int32)
    for q in range(_CHUNK // _LANES):
        k = k_v[pl.ds(q * _LANES, _LANES)]
        rk = jnp.zeros((_LANES,), jnp.int32)
        cnt = jnp.zeros((_LANES,), jnp.int32)
        for l in range(_LANES):
            eq = (k == _lane_splat(k, l)).astype(jnp.int32)
            rk = rk + jnp.where(iota > l, eq, 0)
            cnt = cnt + eq
        prev = plsc.load_gather(cnt_v, [k])
        ranks_v[pl.ds(q * _LANES, _LANES)] = prev + rk
        plsc.addupdate_scatter(cnt_v, [k], cnt, mask=(rk == cnt - 1))
    pltpu.sync_copy(cnt_v, r_hbm.at[wid])
    pltpu.sync_copy(ranks_v, ranks_hbm.at[pl.ds(wid * _CHUNK, _CHUNK)])


def _route_body(inds_hbm, x_hbm, r_hbm, ranks_hbm,
                slot_hbm, texp_hbm, xs_hbm,
                rall_v, btab_v, ttab_v, k_v, rk_v, slot_v, texp_v, xr_v, sem):
    wid = _worker_id()
    iota = lax.iota(jnp.int32, _LANES)
    nq = _C // _LANES
    pltpu.sync_copy(r_hbm, rall_v)
    start = [jnp.zeros((_LANES,), jnp.int32) for _ in range(nq)]
    total = [jnp.zeros((_LANES,), jnp.int32) for _ in range(nq)]
    for w in range(_NW):
        mw = jnp.where(w < wid, jnp.int32(1), jnp.int32(0))
        for q in range(nq):
            v = rall_v[w, pl.ds(q * _LANES, _LANES)]
            total[q] = total[q] + v
            start[q] = start[q] + v * mw
    carry = jnp.zeros((_LANES,), jnp.int32)
    for q in range(nq):
        tiles = (total[q] + (_TT - 1)) >> 6
        incl = jnp.cumsum(tiles)
        excl = incl - tiles + carry
        carry = carry + _lane_splat(incl, _LANES - 1)
        ttab_v[pl.ds(q * _LANES, _LANES)] = excl
        btab_v[pl.ds(q * _LANES, _LANES)] = excl * _TT + start[q]
    pltpu.sync_copy(inds_hbm.at[pl.ds(wid * _CHUNK, _CHUNK)], k_v)
    pltpu.sync_copy(ranks_hbm.at[pl.ds(wid * _CHUNK, _CHUNK)], rk_v)
    for q in range(_CHUNK // _LANES):
        k = k_v[pl.ds(q * _LANES, _LANES)]
        off = plsc.load_gather(btab_v, [k])
        slot_v[pl.ds(q * _LANES, _LANES)] = off + rk_v[pl.ds(q * _LANES, _LANES)]
    pltpu.sync_copy(slot_v, slot_hbm.at[pl.ds(wid * _CHUNK, _CHUNK)])
    pltpu.sync_copy(x_hbm.at[pl.ds(wid * _CHUNK, _CHUNK)], xr_v)
    pltpu.async_copy(xr_v, xs_hbm.at[slot_v], sem).wait()

    @pl.when(wid == 0)
    def _():
        for j in range(_NT // _LANES):
            jv = iota + j * _LANES
            e = jnp.full((_LANES,), -1, jnp.int32)
            for c in range(_C):
                spl = plsc.load_gather(
                    ttab_v, [jnp.full((_LANES,), c, jnp.int32)])
                e = e + (spl <= jv).astype(jnp.int32)
            texp_v[pl.ds(j * _LANES, _LANES)] = e
        pltpu.sync_copy(texp_v, texp_hbm)


def _mm_body(texp_ref, xs_ref, w_ref, b_ref, y_ref):
    i = pl.program_id(0)
    e = texp_ref[i]
    xt = xs_ref[pl.ds(i * _TT, _TT), :].astype(jnp.bfloat16)
    y_ref[pl.ds(i * _TT, _TT), :] = jnp.dot(
        xt, w_ref[e], preferred_element_type=jnp.float32) + b_ref[e]


def _unsort_body(slot_hbm, ys_hbm, out_hbm, slot_v, rows_v, sem):
    wid = _worker_id()
    pltpu.sync_copy(slot_hbm.at[pl.ds(wid * _CHUNK, _CHUNK)], slot_v)
    pltpu.async_copy(ys_hbm.at[slot_v], rows_v, sem).wait()
    pltpu.sync_copy(rows_v, out_hbm.at[pl.ds(wid * _CHUNK, _CHUNK)])


def kernel(x, inds, w, b):
    inds32 = inds.astype(jnp.int32)
    wb = w.astype(jnp.bfloat16)
    mesh = plsc.VectorSubcoreMesh(
        core_axis_name="c", subcore_axis_name="s",
        num_cores=_NC, num_subcores=_NS)

    hist = pl.kernel(
        _hist_body,
        out_type=(
            jax.ShapeDtypeStruct((_NW, _C), jnp.int32),
            jax.ShapeDtypeStruct((_T,), jnp.int32),
        ),
        mesh=mesh,
        compiler_params=pltpu.CompilerParams(needs_layout_passes=False),
        scratch_types=[
            pltpu.VMEM((_CHUNK,), jnp.int32),
            pltpu.VMEM((_C,), jnp.int32),
            pltpu.VMEM((_CHUNK,), jnp.int32),
        ],
    )
    r_tab, ranks = hist(inds32)

    route = pl.kernel(
        _route_body,
        out_type=(
            jax.ShapeDtypeStruct((_T,), jnp.int32),
            jax.ShapeDtypeStruct((_NT,), jnp.int32),
            jax.ShapeDtypeStruct((_PAD, _M), jnp.float32),
        ),
        mesh=mesh,
        compiler_params=pltpu.CompilerParams(needs_layout_passes=False),
        scratch_types=[
            pltpu.VMEM((_NW, _C), jnp.int32),
            pltpu.VMEM((_C,), jnp.int32),
            pltpu.VMEM((_C,), jnp.int32),
            pltpu.VMEM((_CHUNK,), jnp.int32),
            pltpu.VMEM((_CHUNK,), jnp.int32),
            pltpu.VMEM((_CHUNK,), jnp.int32),
            pltpu.VMEM((_NT,), jnp.int32),
            pltpu.VMEM((_CHUNK, _M), jnp.float32),
            pltpu.SemaphoreType.DMA,
        ],
    )
    slot, texp, xs = route(inds32, x, r_tab, ranks)

    ys = pl.pallas_call(
        _mm_body,
        grid_spec=pltpu.PrefetchScalarGridSpec(
            num_scalar_prefetch=1,
            grid=(_NT,),
            in_specs=[
                pl.BlockSpec((_PAD, _M), lambda i, te: (0, 0)),
                pl.BlockSpec((_C, _M, _N), lambda i, te: (0, 0, 0)),
                pl.BlockSpec((_C, 1, _N), lambda i, te: (0, 0, 0)),
            ],
            out_specs=pl.BlockSpec((_PAD, _N), lambda i, te: (0, 0)),
        ),
        out_shape=jax.ShapeDtypeStruct((_PAD, _N), jnp.float32),
    )(texp, xs, wb, b)

    unsort = pl.kernel(
        _unsort_body,
        out_type=jax.ShapeDtypeStruct((_T, _N), jnp.float32),
        mesh=mesh,
        compiler_params=pltpu.CompilerParams(needs_layout_passes=False),
        scratch_types=[
            pltpu.VMEM((_CHUNK,), jnp.int32),
            pltpu.VMEM((_CHUNK, _N), jnp.float32),
            pltpu.SemaphoreType.DMA,
        ],
    )
    return unsort(slot, ys)


# speedup vs baseline: 1.6136x; 1.1475x over previous
import functools

import jax
import jax.numpy as jnp
from jax import lax
from jax.experimental import pallas as pl
from jax.experimental.pallas import tpu as pltpu
from jax.experimental.pallas import tpu_sc as plsc

_C = 64
_M = 256
_N = 256
_T = 2048
_NC = 2
_NS = 16
_NW = _NC * _NS
_CHUNK = _T // _NW
_TT = 64
_NT = _T // _TT + _C // 2
_PAD = _NT * _TT

_LANES = 16
_IOTA = None


def _vgather(v, idx):
    return lax.gather(
        v,
        idx[:, None],
        lax.GatherDimensionNumbers(
            offset_dims=(), collapsed_slice_dims=(0,), start_index_map=(0,)),
        slice_sizes=(1,),
        mode=lax.GatherScatterMode.PROMISE_IN_BOUNDS,
    )


def _lane_splat(v, l):
    return _vgather(v, jnp.full((_LANES,), l, jnp.int32))


def _worker_id():
    return lax.axis_index("s") * _NC + lax.axis_index("c")


def _hist_body(inds_hbm, r_hbm, ranks_hbm, k_v, cnt_v, ranks_v):
    wid = _worker_id()
    pltpu.sync_copy(inds_hbm.at[pl.ds(wid * _CHUNK, _CHUNK)], k_v)
    iota = lax.iota(jnp.int32, _LANES)
    for q in range(_C // _LANES):
        cnt_v[pl.ds(q * _LANES, _LANES)] = jnp.zeros((_LANES,), jnp.int32)
    for q in range(_CHUNK // _LANES):
        k = k_v[pl.ds(q * _LANES, _LANES)]
        rk = jnp.zeros((_LANES,), jnp.int32)
        cnt = jnp.zeros((_LANES,), jnp.int32)
        for l in range(_LANES):
            eq = (k == _lane_splat(k, l)).astype(jnp.int32)
            rk = rk + jnp.where(iota > l, eq, 0)
            cnt = cnt + eq
        prev = plsc.load_gather(cnt_v, [k])
        ranks_v[pl.ds(q * _LANES, _LANES)] = prev + rk
        plsc.addupdate_scatter(cnt_v, [k], cnt, mask=(rk == cnt - 1))
    pltpu.sync_copy(cnt_v, r_hbm.at[wid])
    pltpu.sync_copy(ranks_v, ranks_hbm.at[pl.ds(wid * _CHUNK, _CHUNK)])


def _route_body(inds_hbm, x_hbm, r_hbm, ranks_hbm,
                slot_hbm, texp_hbm, xs_hbm,
                rall_v, btab_v, ttab_v, k_v, rk_v, slot_v, texp_v, xr_v, sem):
    wid = _worker_id()
    iota = lax.iota(jnp.int32, _LANES)
    nq = _C // _LANES
    pltpu.sync_copy(r_hbm, rall_v)
    start = [jnp.zeros((_LANES,), jnp.int32) for _ in range(nq)]
    total = [jnp.zeros((_LANES,), jnp.int32) for _ in range(nq)]
    for w in range(_NW):
        mw = jnp.where(w < wid, jnp.int32(1), jnp.int32(0))
        for q in range(nq):
            v = rall_v[w, pl.ds(q * _LANES, _LANES)]
            total[q] = total[q] + v
            start[q] = start[q] + v * mw
    carry = jnp.zeros((_LANES,), jnp.int32)
    for q in range(nq):
        tiles = (total[q] + (_TT - 1)) >> 6
        incl = jnp.cumsum(tiles)
        excl = incl - tiles + carry
        carry = carry + _lane_splat(incl, _LANES - 1)
        ttab_v[pl.ds(q * _LANES, _LANES)] = excl
        btab_v[pl.ds(q * _LANES, _LANES)] = excl * _TT + start[q]
    pltpu.sync_copy(inds_hbm.at[pl.ds(wid * _CHUNK, _CHUNK)], k_v)
    pltpu.sync_copy(ranks_hbm.at[pl.ds(wid * _CHUNK, _CHUNK)], rk_v)
    for q in range(_CHUNK // _LANES):
        k = k_v[pl.ds(q * _LANES, _LANES)]
        off = plsc.load_gather(btab_v, [k])
        slot_v[pl.ds(q * _LANES, _LANES)] = off + rk_v[pl.ds(q * _LANES, _LANES)]
    pltpu.sync_copy(slot_v, slot_hbm.at[pl.ds(wid * _CHUNK, _CHUNK)])
    pltpu.sync_copy(x_hbm.at[pl.ds(wid * _CHUNK, _CHUNK)], xr_v)
    pltpu.async_copy(xr_v, xs_hbm.at[slot_v], sem).wait()

    @pl.when(wid == 0)
    def _():
        for j in range(_NT // _LANES):
            jv = iota + j * _LANES
            e = jnp.full((_LANES,), -1, jnp.int32)
            for c in range(_C):
                spl = plsc.load_gather(
                    ttab_v, [jnp.full((_LANES,), c, jnp.int32)])
                e = e + (spl <= jv).astype(jnp.int32)
            texp_v[pl.ds(j * _LANES, _LANES)] = e
        pltpu.sync_copy(texp_v, texp_hbm)


_UNROLL = 8


def _mm_body(texp_ref, xs_ref, w_ref, b_ref, y_ref):
    i0 = pl.program_id(0) * _UNROLL
    for u in range(_UNROLL):
        i = i0 + u
        e = texp_ref[i]
        xt = xs_ref[pl.ds(i * _TT, _TT), :].astype(jnp.bfloat16)
        y_ref[pl.ds(i * _TT, _TT), :] = jnp.dot(
            xt, w_ref[e], preferred_element_type=jnp.float32) + b_ref[e]


def _unsort_body(slot_hbm, ys_hbm, out_hbm, slot_v, rows_v, sem):
    wid = _worker_id()
    pltpu.sync_copy(slot_hbm.at[pl.ds(wid * _CHUNK, _CHUNK)], slot_v)
    pltpu.async_copy(ys_hbm.at[slot_v], rows_v, sem).wait()
    pltpu.sync_copy(rows_v, out_hbm.at[pl.ds(wid * _CHUNK, _CHUNK)])


def kernel(x, inds, w, b):
    inds32 = inds.astype(jnp.int32)
    wb = w.astype(jnp.bfloat16)
    mesh = plsc.VectorSubcoreMesh(
        core_axis_name="c", subcore_axis_name="s",
        num_cores=_NC, num_subcores=_NS)

    hist = pl.kernel(
        _hist_body,
        out_type=(
            jax.ShapeDtypeStruct((_NW, _C), jnp.int32),
            jax.ShapeDtypeStruct((_T,), jnp.int32),
        ),
        mesh=mesh,
        compiler_params=pltpu.CompilerParams(needs_layout_passes=False),
        scratch_types=[
            pltpu.VMEM((_CHUNK,), jnp.int32),
            pltpu.VMEM((_C,), jnp.int32),
            pltpu.VMEM((_CHUNK,), jnp.int32),
        ],
    )
    r_tab, ranks = hist(inds32)

    route = pl.kernel(
        _route_body,
        out_type=(
            jax.ShapeDtypeStruct((_T,), jnp.int32),
            jax.ShapeDtypeStruct((_NT,), jnp.int32),
            jax.ShapeDtypeStruct((_PAD, _M), jnp.float32),
        ),
        mesh=mesh,
        compiler_params=pltpu.CompilerParams(needs_layout_passes=False),
        scratch_types=[
            pltpu.VMEM((_NW, _C), jnp.int32),
            pltpu.VMEM((_C,), jnp.int32),
            pltpu.VMEM((_C,), jnp.int32),
            pltpu.VMEM((_CHUNK,), jnp.int32),
            pltpu.VMEM((_CHUNK,), jnp.int32),
            pltpu.VMEM((_CHUNK,), jnp.int32),
            pltpu.VMEM((_NT,), jnp.int32),
            pltpu.VMEM((_CHUNK, _M), jnp.float32),
            pltpu.SemaphoreType.DMA,
        ],
    )
    slot, texp, xs = route(inds32, x, r_tab, ranks)

    ys = pl.pallas_call(
        _mm_body,
        grid_spec=pltpu.PrefetchScalarGridSpec(
            num_scalar_prefetch=1,
            grid=(_NT // _UNROLL,),
            in_specs=[
                pl.BlockSpec((_PAD, _M), lambda i, te: (0, 0)),
                pl.BlockSpec((_C, _M, _N), lambda i, te: (0, 0, 0)),
                pl.BlockSpec((_C, 1, _N), lambda i, te: (0, 0, 0)),
            ],
            out_specs=pl.BlockSpec((_PAD, _N), lambda i, te: (0, 0)),
        ),
        out_shape=jax.ShapeDtypeStruct((_PAD, _N), jnp.float32),
    )(texp, xs, wb, b)

    unsort = pl.kernel(
        _unsort_body,
        out_type=jax.ShapeDtypeStruct((_T, _N), jnp.float32),
        mesh=mesh,
        compiler_params=pltpu.CompilerParams(needs_layout_passes=False),
        scratch_types=[
            pltpu.VMEM((_CHUNK,), jnp.int32),
            pltpu.VMEM((_CHUNK, _N), jnp.float32),
            pltpu.SemaphoreType.DMA,
        ],
    )
    return unsort(slot, ys)
